# baseline (speedup 1.0000x reference)
import functools

import jax
import jax.numpy as jnp
from jax import lax
from jax.experimental import pallas as pl
from jax.experimental.pallas import tpu as pltpu
from jax.experimental.pallas import tpu_sc as plsc

N = 16384
NEWS_DIM = 64
CAT_DIM = 16
D = NEWS_DIM + CAT_DIM
NUM_NEWS = 1000000
NUM_CATEGORIES = 1000
CH = 128


@functools.lru_cache(maxsize=1)
def _build_sc_kernel():
    info = plsc.get_sparse_core_info()
    nc, ns = info.num_cores, info.num_subcores
    nw = nc * ns
    bpw = N // nw
    n_chunks = bpw // CH
    cat_words = NUM_CATEGORIES * CAT_DIM
    mesh = plsc.VectorSubcoreMesh(core_axis_name="c", subcore_axis_name="s")

    @functools.partial(
        pl.kernel,
        mesh=mesh,
        out_type=jax.ShapeDtypeStruct((N * D,), jnp.float32),
        scratch_types=[
            pltpu.VMEM((bpw,), jnp.int32),
            pltpu.VMEM((bpw,), jnp.int32),
            pltpu.VMEM((bpw,), jnp.float32),
            pltpu.VMEM((n_chunks, CH), jnp.int32),
            pltpu.VMEM((bpw, 2 * NEWS_DIM), jnp.float32),
            pltpu.VMEM((cat_words,), jnp.float32),
            pltpu.VMEM((bpw * D,), jnp.float32),
            pltpu.SemaphoreType.DMA,
        ],
    )
    def sc_kernel(news_ids_hbm, cat_ids_hbm, mask_hbm, news_tab_hbm,
                  cat_tab_hbm, out_hbm,
                  nids_v, cids_v, mask_v, nidx_v, news_v, cat_tab_v, out_v,
                  sem):
        wid = lax.axis_index("s") * nc + lax.axis_index("c")
        base = wid * bpw
        pltpu.sync_copy(news_ids_hbm.at[pl.ds(base, bpw)], nids_v)
        pltpu.sync_copy(cat_ids_hbm.at[pl.ds(base, bpw)], cids_v)
        pltpu.sync_copy(mask_hbm.at[pl.ds(base, bpw)], mask_v)

        for c in range(n_chunks):
            for v in range(CH // 16):
                nidx_v[c, pl.ds(v * 16, 16)] = lax.shift_right_logical(
                    nids_v[pl.ds(c * CH + v * 16, 16)], 1)

        copies = [pltpu.async_copy(cat_tab_hbm, cat_tab_v, sem)]
        for c in range(n_chunks):
            copies.append(pltpu.async_copy(
                news_tab_hbm.at[nidx_v.at[c]],
                news_v.at[pl.ds(c * CH, CH)], sem))
        for c in copies:
            c.wait()

        def body(g, carry):
            ids16 = nids_v[pl.ds(g * 16, 16)]
            off16 = (ids16 & 1) * NEWS_DIM
            cpos16 = cids_v[pl.ds(g * 16, 16)] * CAT_DIM
            m16 = mask_v[pl.ds(g * 16, 16)]
            for k in range(16):
                i = g * 16 + k
                off = off16[k]
                cp = cpos16[k]
                m = m16[k]
                obase = i * D
                for j in range(NEWS_DIM // 16):
                    out_v[pl.ds(obase + j * 16, 16)] = (
                        news_v[i, pl.ds(off + j * 16, 16)] * m)
                out_v[pl.ds(obase + NEWS_DIM, CAT_DIM)] = (
                    cat_tab_v[pl.ds(cp, CAT_DIM)] * m)
            return carry
        lax.fori_loop(0, bpw // 16, body, 0)

        pltpu.sync_copy(out_v, out_hbm.at[pl.ds(base * D, bpw * D)])

    return sc_kernel


def kernel(news_ids, category_ids, delta_t, mask, news_table, category_table):
    sc = _build_sc_kernel()
    news2 = jnp.reshape(news_table, (NUM_NEWS // 2, 2 * NEWS_DIM))
    cat_flat = jnp.reshape(category_table, (NUM_CATEGORIES * CAT_DIM,))
    out = sc(news_ids, category_ids, mask, news2, cat_flat)
    return (jnp.reshape(out, (N, D)), delta_t)


# speedup vs baseline: 1.6596x; 1.6596x over previous
import functools

import jax
import jax.numpy as jnp
from jax import lax
from jax.experimental import pallas as pl
from jax.experimental.pallas import tpu as pltpu
from jax.experimental.pallas import tpu_sc as plsc

N = 16384
NEWS_DIM = 64
CAT_DIM = 16
D = NEWS_DIM + CAT_DIM
NUM_NEWS = 1000000
NUM_CATEGORIES = 1000


@functools.lru_cache(maxsize=1)
def _build_sc_kernel():
    info = plsc.get_sparse_core_info()
    nc, ns = info.num_cores, info.num_subcores
    nw = nc * ns
    bpw = N // nw
    cat_words = NUM_CATEGORIES * CAT_DIM
    mesh = plsc.VectorSubcoreMesh(core_axis_name="c", subcore_axis_name="s")

    @functools.partial(
        pl.kernel,
        mesh=mesh,
        out_type=jax.ShapeDtypeStruct((N * D,), jnp.float32),
        scratch_types=[
            pltpu.VMEM((bpw,), jnp.int32),
            pltpu.VMEM((bpw,), jnp.int32),
            pltpu.VMEM((bpw,), jnp.float32),
            pltpu.VMEM((bpw, NEWS_DIM), jnp.float32),
            pltpu.VMEM((cat_words,), jnp.float32),
            pltpu.VMEM((bpw * D,), jnp.float32),
            pltpu.SemaphoreType.DMA,
            pltpu.SemaphoreType.DMA,
        ],
    )
    def sc_kernel(news_ids_hbm, cat_ids_hbm, mask_hbm, news_tab_hbm,
                  cat_tab_hbm, out_hbm,
                  nids_v, cids_v, mask_v, news_v, cat_tab_v, out_v,
                  sem, csem):
        wid = lax.axis_index("s") * nc + lax.axis_index("c")
        base = wid * bpw
        pltpu.sync_copy(news_ids_hbm.at[pl.ds(base, bpw)], nids_v)
        pltpu.sync_copy(cat_ids_hbm.at[pl.ds(base, bpw)], cids_v)
        pltpu.sync_copy(mask_hbm.at[pl.ds(base, bpw)], mask_v)
        ccopy = pltpu.async_copy(cat_tab_hbm, cat_tab_v, csem)

        def fire(g, carry):
            ids16 = nids_v[pl.ds(g * 16, 16)]
            for k in range(16):
                i = g * 16 + k
                r = ids16[k]
                pltpu.async_copy(news_tab_hbm.at[pl.ds(r, 1)],
                                 news_v.at[pl.ds(i, 1)], sem)
            return carry
        lax.fori_loop(0, bpw // 16, fire, 0)
        pltpu.make_async_copy(news_tab_hbm.at[pl.ds(0, bpw)], news_v,
                              sem).wait()
        ccopy.wait()

        def body(g, carry):
            cpos16 = cids_v[pl.ds(g * 16, 16)] * CAT_DIM
            m16 = mask_v[pl.ds(g * 16, 16)]
            for k in range(16):
                i = g * 16 + k
                cp = cpos16[k]
                m = m16[k]
                obase = i * D
                for j in range(NEWS_DIM // 16):
                    out_v[pl.ds(obase + j * 16, 16)] = (
                        news_v[i, pl.ds(j * 16, 16)] * m)
                out_v[pl.ds(obase + NEWS_DIM, CAT_DIM)] = (
                    cat_tab_v[pl.ds(cp, CAT_DIM)] * m)
            return carry
        lax.fori_loop(0, bpw // 16, body, 0)

        pltpu.sync_copy(out_v, out_hbm.at[pl.ds(base * D, bpw * D)])

    return sc_kernel


def kernel(news_ids, category_ids, delta_t, mask, news_table, category_table):
    sc = _build_sc_kernel()
    cat_flat = jnp.reshape(category_table, (NUM_CATEGORIES * CAT_DIM,))
    out = sc(news_ids, category_ids, mask, news_table, cat_flat)
    return (jnp.reshape(out, (N, D)), delta_t)


# speedup vs baseline: 1.6597x; 1.0000x over previous
import functools

import jax
import jax.numpy as jnp
from jax import lax
from jax.experimental import pallas as pl
from jax.experimental.pallas import tpu as pltpu
from jax.experimental.pallas import tpu_sc as plsc

N = 16384
NEWS_DIM = 64
CAT_DIM = 16
D = NEWS_DIM + CAT_DIM
NUM_NEWS = 1000000
NUM_CATEGORIES = 1000


@functools.lru_cache(maxsize=1)
def _build_sc_kernel():
    info = plsc.get_sparse_core_info()
    nc, ns = info.num_cores, info.num_subcores
    nw = nc * ns
    bpw = N // nw
    cat_words = NUM_CATEGORIES * CAT_DIM
    mesh = plsc.VectorSubcoreMesh(core_axis_name="c", subcore_axis_name="s")

    @functools.partial(
        pl.kernel,
        mesh=mesh,
        out_type=jax.ShapeDtypeStruct((N * D,), jnp.float32),
        scratch_types=[
            pltpu.VMEM((bpw,), jnp.int32),
            pltpu.VMEM((bpw,), jnp.int32),
            pltpu.VMEM((bpw,), jnp.float32),
            pltpu.VMEM((bpw, NEWS_DIM), jnp.float32),
            pltpu.VMEM((cat_words,), jnp.float32),
            pltpu.VMEM((bpw * D,), jnp.float32),
            pltpu.SemaphoreType.DMA,
            pltpu.SemaphoreType.DMA,
            pltpu.SemaphoreType.DMA,
            pltpu.SemaphoreType.DMA,
            pltpu.SemaphoreType.DMA,
            pltpu.SemaphoreType.DMA,
            pltpu.SemaphoreType.DMA,
            pltpu.SemaphoreType.DMA,
            pltpu.SemaphoreType.DMA,
        ],
    )
    def sc_kernel(news_ids_hbm, cat_ids_hbm, mask_hbm, news_tab_hbm,
                  cat_tab_hbm, out_hbm,
                  nids_v, cids_v, mask_v, news_v, cat_tab_v, out_v,
                  sem0, sem1, sem2, sem3, sem4, sem5, sem6, sem7, csem):
        wid = lax.axis_index("s") * nc + lax.axis_index("c")
        base = wid * bpw
        pltpu.sync_copy(news_ids_hbm.at[pl.ds(base, bpw)], nids_v)
        pltpu.sync_copy(cat_ids_hbm.at[pl.ds(base, bpw)], cids_v)
        pltpu.sync_copy(mask_hbm.at[pl.ds(base, bpw)], mask_v)
        ccopy = pltpu.async_copy(cat_tab_hbm, cat_tab_v, csem)

        sems = (sem0, sem1, sem2, sem3, sem4, sem5, sem6, sem7)

        def fire(g, carry):
            ids16 = nids_v[pl.ds(g * 16, 16)]
            for k in range(16):
                i = g * 16 + k
                r = ids16[k]
                pltpu.async_copy(news_tab_hbm.at[pl.ds(r, 1)],
                                 news_v.at[pl.ds(i, 1)], sems[k % 8])
            return carry
        lax.fori_loop(0, bpw // 16, fire, 0)
        per_sem = bpw // 8
        for q in range(8):
            pltpu.make_async_copy(news_tab_hbm.at[pl.ds(0, per_sem)],
                                  news_v.at[pl.ds(0, per_sem)],
                                  sems[q]).wait()
        ccopy.wait()

        def body(g, carry):
            cpos16 = cids_v[pl.ds(g * 16, 16)] * CAT_DIM
            m16 = mask_v[pl.ds(g * 16, 16)]
            for k in range(16):
                i = g * 16 + k
                cp = cpos16[k]
                m = m16[k]
                obase = i * D
                for j in range(NEWS_DIM // 16):
                    out_v[pl.ds(obase + j * 16, 16)] = (
                        news_v[i, pl.ds(j * 16, 16)] * m)
                out_v[pl.ds(obase + NEWS_DIM, CAT_DIM)] = (
                    cat_tab_v[pl.ds(cp, CAT_DIM)] * m)
            return carry
        lax.fori_loop(0, bpw // 16, body, 0)

        pltpu.sync_copy(out_v, out_hbm.at[pl.ds(base * D, bpw * D)])

    return sc_kernel


def kernel(news_ids, category_ids, delta_t, mask, news_table, category_table):
    sc = _build_sc_kernel()
    cat_flat = jnp.reshape(category_table, (NUM_CATEGORIES * CAT_DIM,))
    out = sc(news_ids, category_ids, mask, news_table, cat_flat)
    return (jnp.reshape(out, (N, D)), delta_t)
